# baseline (speedup 1.0000x reference)
import functools

import jax
import jax.numpy as jnp
from jax import lax
from jax.experimental import pallas as pl
from jax.experimental.pallas import tpu as pltpu
from jax.experimental.pallas import tpu_sc as plsc

N = 10000
E = 320000
F = 16
NCLS = 40
NC, NS, L = 2, 16, 16
NW = NC * NS
CH = 80
NCHUNK = 125
EPW = NCHUNK * CH
EPAD = NW * EPW
NP = 10240
ROWS_T = NP // NS
DEGP = 10240
DEG_T = DEGP // NS

NBUF = 10
DEPTH = 5
MAIN = 120

_MESH = plsc.VectorSubcoreMesh(core_axis_name="c", subcore_axis_name="s")
_SC_PARAMS = pltpu.CompilerParams(
    use_tc_tiling_on_sc=False,
    disable_bounds_checks=True,
    disable_semaphore_checks=True,
)

_SPLAT_DNUMS = lax.GatherDimensionNumbers(
    offset_dims=(), collapsed_slice_dims=(0,), start_index_map=(0,))


def _splat(v, k):
    idx = jnp.full((L, 1), k, jnp.int32)
    return lax.gather(v, idx, _SPLAT_DNUMS, (1,),
                      mode=lax.GatherScatterMode.PROMISE_IN_BOUNDS)


@functools.partial(
    pl.kernel,
    out_type=jax.ShapeDtypeStruct((NC, DEGP, F), jnp.float32),
    mesh=_MESH,
    compiler_params=_SC_PARAMS,
    scratch_types=[
        pltpu.VMEM((NCHUNK, CH), jnp.int32),
        pltpu.VMEM((NCHUNK, CH), jnp.float32),
        pltpu.VMEM((DEG_T,), jnp.float32),
        pltpu.VMEM((DEG_T, F), jnp.float32),
        pltpu.VMEM_SHARED((DEGP,), jnp.float32),
        pltpu.SemaphoreType.DMA,
    ],
)
def _deg_kernel(col_hbm, ew_hbm, out_hbm, col_v, ew_v, dbuf, wbuf, acc, sem):
    c = lax.axis_index("c")
    s = lax.axis_index("s")
    wid = c * NS + s
    pltpu.sync_copy(col_hbm.at[wid], col_v)
    pltpu.sync_copy(ew_hbm.at[wid], ew_v)

    @pl.loop(0, DEG_T // L)
    def _z(i):
        dbuf[pl.ds(i * L, L)] = jnp.zeros((L,), jnp.float32)

    pltpu.sync_copy(dbuf, acc.at[pl.ds(s * DEG_T, DEG_T)])
    plsc.subcore_barrier()

    @pl.loop(0, NCHUNK)
    def _chunk(j):
        pltpu.sync_copy(ew_v.at[j], acc.at[col_v.at[j]], add=True)

    plsc.subcore_barrier()
    pltpu.sync_copy(acc.at[pl.ds(s * DEG_T, DEG_T)], dbuf)
    for r in range(DEG_T):
        wbuf[r, :] = jnp.ones((L,), jnp.float32)
    for g in range(DEG_T // L):
        v = dbuf[pl.ds(g * L, L)]
        for k in range(L):
            r = g * L + k
            wbuf[r, :] = wbuf[r, :] * _splat(v, k)
    pltpu.sync_copy(wbuf, out_hbm.at[c, pl.ds(s * DEG_T, DEG_T), :])


@functools.partial(
    pl.kernel,
    out_type=jax.ShapeDtypeStruct((NC, NP, F), jnp.float32),
    mesh=_MESH,
    compiler_params=_SC_PARAMS,
    scratch_types=[
        pltpu.VMEM((NCHUNK, CH), jnp.int32),
        pltpu.VMEM((NCHUNK, CH), jnp.int32),
        pltpu.VMEM((NCHUNK, CH), jnp.float32),
        [pltpu.VMEM((CH, F), jnp.float32) for _ in range(NBUF)],
        pltpu.VMEM((ROWS_T, F), jnp.float32),
        pltpu.VMEM_SHARED((NP, F), jnp.float32),
        [pltpu.SemaphoreType.DMA for _ in range(NBUF)],
        [pltpu.SemaphoreType.DMA for _ in range(NBUF)],
        pltpu.SemaphoreType.DMA,
    ],
)
def _agg_kernel(y_hbm, row_hbm, col_hbm, ew_hbm, out_hbm,
                row_v, col_v, ew_v, rbufs, tbuf, acc, gsems, ssems, stsem):
    c = lax.axis_index("c")
    s = lax.axis_index("s")
    wid = c * NS + s
    pltpu.async_copy(row_hbm.at[wid], row_v, stsem)
    pltpu.async_copy(col_hbm.at[wid], col_v, stsem)
    pltpu.async_copy(ew_hbm.at[wid], ew_v, stsem)

    @pl.loop(0, ROWS_T // 8)
    def _z(i):
        for r in range(8):
            tbuf[i * 8 + r, :] = jnp.zeros((L,), jnp.float32)

    pltpu.sync_copy(tbuf, acc.at[pl.ds(s * ROWS_T, ROWS_T)])
    pltpu.make_async_copy(row_hbm.at[wid], row_v, stsem).wait()
    pltpu.make_async_copy(col_hbm.at[wid], col_v, stsem).wait()
    pltpu.make_async_copy(ew_hbm.at[wid], ew_v, stsem).wait()
    plsc.subcore_barrier()

    def gather_start(j, b):
        pltpu.async_copy(y_hbm.at[row_v.at[j]], rbufs[b], gsems[b])

    def gather_wait(j, b):
        pltpu.make_async_copy(y_hbm.at[row_v.at[j]], rbufs[b], gsems[b]).wait()

    def scatter_start(j, b):
        pltpu.async_copy(rbufs[b], acc.at[col_v.at[j]], ssems[b], add=True)

    def scatter_wait(j, b):
        pltpu.make_async_copy(rbufs[b], acc.at[col_v.at[j]], ssems[b]).wait()

    def scale(j, b):
        @pl.loop(0, CH // L)
        def _sg(g):
            sv = ew_v[j, pl.ds(g * L, L)]
            for k in range(L):
                rbufs[b][g * L + k, :] = rbufs[b][g * L + k, :] * _splat(sv, k)

    for b in range(DEPTH):
        gather_start(b, b)

    @pl.loop(0, MAIN // NBUF)
    def _grp(gi):
        for b in range(NBUF):
            j = gi * NBUF + b
            gather_wait(j, b)
            scale(j, b)
            scatter_start(j, b)
            b2 = (b + DEPTH) % NBUF
            jw = j - (NBUF - DEPTH)

            @pl.when(jw >= 0)
            def _w():
                scatter_wait(jw, b2)

            gather_start(j + DEPTH, b2)

    for j in range(MAIN, NCHUNK):
        b = j % NBUF
        gather_wait(j, b)
        scale(j, b)
        scatter_start(j, b)
    for j in range(NCHUNK - NBUF, NCHUNK):
        scatter_wait(j, j % NBUF)

    plsc.subcore_barrier()
    pltpu.sync_copy(acc.at[pl.ds(s * ROWS_T, ROWS_T)], tbuf)
    pltpu.sync_copy(tbuf, out_hbm.at[c, pl.ds(s * ROWS_T, ROWS_T), :])


def _prep_body(degp_ref, x_ref, w_ref, dinv_ref, y_ref):
    deg = degp_ref[0, :N, :] + degp_ref[1, :N, :] + 1.0
    dinv = 1.0 / jnp.sqrt(deg)
    dinv_ref[...] = dinv
    xw = jnp.dot(x_ref[...], w_ref[...], preferred_element_type=jnp.float32)
    y_ref[...] = dinv * xw


_prep = pl.pallas_call(
    _prep_body,
    out_shape=(jax.ShapeDtypeStruct((N, F), jnp.float32),
               jax.ShapeDtypeStruct((N, F), jnp.float32)))


def _mid_body(aggp_ref, y_ref, dinv_ref, b_ref, w_ref, o_ref):
    agg = aggp_ref[0, :N, :] + aggp_ref[1, :N, :]
    h = dinv_ref[...] * (agg + y_ref[...]) + b_ref[...]
    h = jnp.maximum(h, 0.0)
    o_ref[...] = dinv_ref[...] * jnp.dot(h, w_ref[...],
                                         preferred_element_type=jnp.float32)


_mid = pl.pallas_call(
    _mid_body, out_shape=jax.ShapeDtypeStruct((N, F), jnp.float32))


def _mid2_body(aggp_ref, y_ref, dinv_ref, b_ref, o_ref):
    agg = aggp_ref[0, :N, :] + aggp_ref[1, :N, :]
    h = dinv_ref[...] * (agg + y_ref[...]) + b_ref[...]
    o_ref[...] = dinv_ref[...] * jnp.maximum(h, 0.0)


_mid2 = pl.pallas_call(
    _mid2_body, out_shape=jax.ShapeDtypeStruct((N, F), jnp.float32))


def _final_body(aggp_ref, y_ref, dinv_ref, b_ref, w_ref, o_ref):
    agg = aggp_ref[0, :N, :] + aggp_ref[1, :N, :]
    a = dinv_ref[...] * (agg + y_ref[...])
    o = jnp.dot(a, w_ref[...], preferred_element_type=jnp.float32) + b_ref[...]
    m = jnp.max(o, axis=1, keepdims=True)
    lse = jnp.log(jnp.sum(jnp.exp(o - m), axis=1, keepdims=True)) + m
    o_ref[...] = o - lse


_final = pl.pallas_call(
    _final_body, out_shape=jax.ShapeDtypeStruct((N, NCLS), jnp.float32))


def kernel(x, edge_index, edge_weight, W1, b1, W3, b3, W2, b2):
    pad = EPAD - E
    zi = jnp.zeros((pad,), edge_index.dtype)
    row3 = jnp.concatenate([edge_index[0], zi]).reshape(NW, NCHUNK, CH)
    col3 = jnp.concatenate([edge_index[1], zi]).reshape(NW, NCHUNK, CH)
    ew3 = jnp.concatenate(
        [edge_weight, jnp.zeros((pad,), edge_weight.dtype)]
    ).reshape(NW, NCHUNK, CH)
    degp = _deg_kernel(col3, ew3)
    dinv16, y1 = _prep(degp, x, W1)
    a1 = _agg_kernel(y1, row3, col3, ew3)
    y2 = _mid(a1, y1, dinv16, b1.reshape(1, F), W3)
    a2 = _agg_kernel(y2, row3, col3, ew3)
    y3 = _mid2(a2, y2, dinv16, b3.reshape(1, F))
    a3 = _agg_kernel(y3, row3, col3, ew3)
    return _final(a3, y3, dinv16, b2.reshape(1, NCLS), W2)


# speedup vs baseline: 1.0631x; 1.0631x over previous
import functools

import jax
import jax.numpy as jnp
from jax import lax
from jax.experimental import pallas as pl
from jax.experimental.pallas import tpu as pltpu
from jax.experimental.pallas import tpu_sc as plsc

N = 10000
E = 320000
F = 16
NCLS = 40
NC, NS, L = 2, 16, 16
NW = NC * NS
CH = 80
NCHUNK = 125
EPW = NCHUNK * CH
EPAD = NW * EPW
NP = 10240
ROWS_T = NP // NS
DEGP = 10240
DEG_T = DEGP // NS

NBUF = 8
DEPTH = 5
MAIN = 120

_MESH = plsc.VectorSubcoreMesh(core_axis_name="c", subcore_axis_name="s")
_SC_PARAMS = pltpu.CompilerParams(
    use_tc_tiling_on_sc=False,
    disable_bounds_checks=True,
    disable_semaphore_checks=True,
)

_SPLAT_DNUMS = lax.GatherDimensionNumbers(
    offset_dims=(), collapsed_slice_dims=(0,), start_index_map=(0,))


def _splat(v, k):
    idx = jnp.full((L, 1), k, jnp.int32)
    return lax.gather(v, idx, _SPLAT_DNUMS, (1,),
                      mode=lax.GatherScatterMode.PROMISE_IN_BOUNDS)


DWIN = 8


@functools.partial(
    pl.kernel,
    out_type=(jax.ShapeDtypeStruct((NC, NP, F), jnp.float32),
              jax.ShapeDtypeStruct((DEGP, F), jnp.float32),
              jax.ShapeDtypeStruct((NP, F), jnp.float32)),
    mesh=_MESH,
    compiler_params=_SC_PARAMS,
    scratch_types=[
        pltpu.VMEM((NCHUNK, CH), jnp.int32),
        pltpu.VMEM((NCHUNK, CH), jnp.int32),
        pltpu.VMEM((NCHUNK, CH), jnp.int32),
        pltpu.VMEM((NCHUNK, CH), jnp.float32),
        pltpu.VMEM((NCHUNK, CH), jnp.float32),
        [pltpu.VMEM((CH, F), jnp.float32) for _ in range(NBUF)],
        pltpu.VMEM((ROWS_T, F), jnp.float32),
        pltpu.VMEM((ROWS_T, F), jnp.float32),
        pltpu.VMEM((ROWS_T, F), jnp.float32),
        pltpu.VMEM((DEG_T,), jnp.float32),
        pltpu.VMEM_SHARED((DEGP,), jnp.float32),
        pltpu.VMEM_SHARED((NP, F), jnp.float32),
        [pltpu.SemaphoreType.DMA for _ in range(NBUF)],
        [pltpu.SemaphoreType.DMA for _ in range(NBUF)],
        pltpu.SemaphoreType.DMA,
        pltpu.SemaphoreType.DMA,
        pltpu.SemaphoreType.DMA,
    ],
)
def _layer1_kernel(xw_hbm, row_hbm, col_hbm, ew_hbm,
                   out_hbm, dinv_hbm, y_hbm,
                   row_v, colA_v, colB_v, ewA_v, ewB_v, rbufs,
                   tbuf, xwbuf, wbuf, dbuf, dacc, acc,
                   gsems, ssems, stsem, dsemA, dsemB):
    c = lax.axis_index("c")
    s = lax.axis_index("s")
    wid = c * NS + s
    mid = (1 - c) * NS + s
    pltpu.async_copy(row_hbm.at[wid], row_v, stsem)
    pltpu.async_copy(col_hbm.at[wid], colA_v, stsem)
    pltpu.async_copy(ew_hbm.at[wid], ewA_v, stsem)
    pltpu.async_copy(col_hbm.at[mid], colB_v, stsem)
    pltpu.async_copy(ew_hbm.at[mid], ewB_v, stsem)
    pltpu.async_copy(xw_hbm.at[pl.ds(s * ROWS_T, ROWS_T), :], xwbuf, stsem)

    @pl.loop(0, ROWS_T // 8)
    def _z(i):
        for r in range(8):
            tbuf[i * 8 + r, :] = jnp.zeros((L,), jnp.float32)

    @pl.loop(0, DEG_T // L)
    def _zd(i):
        dbuf[pl.ds(i * L, L)] = jnp.zeros((L,), jnp.float32)

    pltpu.sync_copy(tbuf, acc.at[pl.ds(s * ROWS_T, ROWS_T)])
    pltpu.sync_copy(dbuf, dacc.at[pl.ds(s * DEG_T, DEG_T)])
    pltpu.make_async_copy(row_hbm.at[wid], row_v, stsem).wait()
    pltpu.make_async_copy(col_hbm.at[wid], colA_v, stsem).wait()
    pltpu.make_async_copy(ew_hbm.at[wid], ewA_v, stsem).wait()
    pltpu.make_async_copy(col_hbm.at[mid], colB_v, stsem).wait()
    pltpu.make_async_copy(ew_hbm.at[mid], ewB_v, stsem).wait()
    plsc.subcore_barrier()

    def dfire(j):
        pltpu.async_copy(ewA_v.at[j], dacc.at[colA_v.at[j]], dsemA, add=True)
        pltpu.async_copy(ewB_v.at[j], dacc.at[colB_v.at[j]], dsemB, add=True)

    def dwait(j):
        pltpu.make_async_copy(ewA_v.at[j], dacc.at[colA_v.at[j]], dsemA).wait()
        pltpu.make_async_copy(ewB_v.at[j], dacc.at[colB_v.at[j]], dsemB).wait()

    for j in range(DWIN):
        dfire(j)

    @pl.loop(DWIN, NCHUNK)
    def _dchunk(j):
        dwait(j - DWIN)
        dfire(j)

    for j in range(NCHUNK - DWIN, NCHUNK):
        dwait(j)

    plsc.subcore_barrier()
    pltpu.sync_copy(dacc.at[pl.ds(s * DEG_T, DEG_T)], dbuf)

    @pl.loop(0, DEG_T // L)
    def _nr(g):
        d = dbuf[pl.ds(g * L, L)] + 1.0
        i = lax.bitcast_convert_type(d, jnp.int32)
        i = jnp.full((L,), 0x5F3759DF, jnp.int32) - (i >> 1)
        y = lax.bitcast_convert_type(i, jnp.float32)
        hd = d * 0.5
        for _ in range(3):
            y = y * (1.5 - hd * y * y)
        dbuf[pl.ds(g * L, L)] = y

    @pl.loop(0, ROWS_T // 8)
    def _o(i):
        for r in range(8):
            wbuf[i * 8 + r, :] = jnp.ones((L,), jnp.float32)

    pltpu.make_async_copy(xw_hbm.at[pl.ds(s * ROWS_T, ROWS_T), :], xwbuf,
                          stsem).wait()

    @pl.loop(0, DEG_T // L)
    def _ex(g):
        v = dbuf[pl.ds(g * L, L)]
        for k in range(L):
            r = g * L + k
            spl = _splat(v, k)
            wbuf[r, :] = wbuf[r, :] * spl
            xwbuf[r, :] = xwbuf[r, :] * spl
    pltpu.sync_copy(wbuf, dinv_hbm.at[pl.ds(s * DEG_T, DEG_T), :])
    pltpu.sync_copy(xwbuf, y_hbm.at[pl.ds(s * ROWS_T, ROWS_T), :])
    plsc.subcore_barrier()

    def gather_start(j, b):
        pltpu.async_copy(y_hbm.at[row_v.at[j]], rbufs[b], gsems[b])

    def gather_wait(j, b):
        pltpu.make_async_copy(y_hbm.at[row_v.at[j]], rbufs[b], gsems[b]).wait()

    def scatter_start(j, b):
        pltpu.async_copy(rbufs[b], acc.at[colA_v.at[j]], ssems[b], add=True)

    def scatter_wait(j, b):
        pltpu.make_async_copy(rbufs[b], acc.at[colA_v.at[j]], ssems[b]).wait()

    def scale(j, b):
        @pl.loop(0, CH // L)
        def _sg(g):
            sv = ewA_v[j, pl.ds(g * L, L)]
            for k in range(L):
                rbufs[b][g * L + k, :] = rbufs[b][g * L + k, :] * _splat(sv, k)

    for b in range(DEPTH):
        gather_start(b, b)

    @pl.loop(0, MAIN // NBUF)
    def _grp(gi):
        for b in range(NBUF):
            j = gi * NBUF + b
            gather_wait(j, b)
            scale(j, b)
            scatter_start(j, b)
            b2 = (b + DEPTH) % NBUF
            jw = j - (NBUF - DEPTH)

            @pl.when(jw >= 0)
            def _w():
                scatter_wait(jw, b2)

            gather_start(j + DEPTH, b2)

    for j in range(MAIN, NCHUNK):
        b = j % NBUF
        gather_wait(j, b)
        scale(j, b)
        scatter_start(j, b)
    for j in range(NCHUNK - NBUF, NCHUNK):
        scatter_wait(j, j % NBUF)

    plsc.subcore_barrier()
    pltpu.sync_copy(acc.at[pl.ds(s * ROWS_T, ROWS_T)], tbuf)
    pltpu.sync_copy(tbuf, out_hbm.at[c, pl.ds(s * ROWS_T, ROWS_T), :])


@functools.partial(
    pl.kernel,
    out_type=jax.ShapeDtypeStruct((NC, NP, F), jnp.float32),
    mesh=_MESH,
    compiler_params=_SC_PARAMS,
    scratch_types=[
        pltpu.VMEM((NCHUNK, CH), jnp.int32),
        pltpu.VMEM((NCHUNK, CH), jnp.int32),
        pltpu.VMEM((NCHUNK, CH), jnp.float32),
        [pltpu.VMEM((CH, F), jnp.float32) for _ in range(NBUF)],
        pltpu.VMEM((ROWS_T, F), jnp.float32),
        pltpu.VMEM_SHARED((NP, F), jnp.float32),
        [pltpu.SemaphoreType.DMA for _ in range(NBUF)],
        [pltpu.SemaphoreType.DMA for _ in range(NBUF)],
        pltpu.SemaphoreType.DMA,
    ],
)
def _agg_kernel(y_hbm, row_hbm, col_hbm, ew_hbm, out_hbm,
                row_v, col_v, ew_v, rbufs, tbuf, acc, gsems, ssems, stsem):
    c = lax.axis_index("c")
    s = lax.axis_index("s")
    wid = c * NS + s
    pltpu.async_copy(row_hbm.at[wid], row_v, stsem)
    pltpu.async_copy(col_hbm.at[wid], col_v, stsem)
    pltpu.async_copy(ew_hbm.at[wid], ew_v, stsem)

    @pl.loop(0, ROWS_T // 8)
    def _z(i):
        for r in range(8):
            tbuf[i * 8 + r, :] = jnp.zeros((L,), jnp.float32)

    pltpu.sync_copy(tbuf, acc.at[pl.ds(s * ROWS_T, ROWS_T)])
    pltpu.make_async_copy(row_hbm.at[wid], row_v, stsem).wait()
    pltpu.make_async_copy(col_hbm.at[wid], col_v, stsem).wait()
    pltpu.make_async_copy(ew_hbm.at[wid], ew_v, stsem).wait()
    plsc.subcore_barrier()

    def gather_start(j, b):
        pltpu.async_copy(y_hbm.at[row_v.at[j]], rbufs[b], gsems[b])

    def gather_wait(j, b):
        pltpu.make_async_copy(y_hbm.at[row_v.at[j]], rbufs[b], gsems[b]).wait()

    def scatter_start(j, b):
        pltpu.async_copy(rbufs[b], acc.at[col_v.at[j]], ssems[b], add=True)

    def scatter_wait(j, b):
        pltpu.make_async_copy(rbufs[b], acc.at[col_v.at[j]], ssems[b]).wait()

    def scale(j, b):
        @pl.loop(0, CH // L)
        def _sg(g):
            sv = ew_v[j, pl.ds(g * L, L)]
            for k in range(L):
                rbufs[b][g * L + k, :] = rbufs[b][g * L + k, :] * _splat(sv, k)

    for b in range(DEPTH):
        gather_start(b, b)

    @pl.loop(0, MAIN // NBUF)
    def _grp(gi):
        for b in range(NBUF):
            j = gi * NBUF + b
            gather_wait(j, b)
            scale(j, b)
            scatter_start(j, b)
            b2 = (b + DEPTH) % NBUF
            jw = j - (NBUF - DEPTH)

            @pl.when(jw >= 0)
            def _w():
                scatter_wait(jw, b2)

            gather_start(j + DEPTH, b2)

    for j in range(MAIN, NCHUNK):
        b = j % NBUF
        gather_wait(j, b)
        scale(j, b)
        scatter_start(j, b)
    for j in range(NCHUNK - NBUF, NCHUNK):
        scatter_wait(j, j % NBUF)

    plsc.subcore_barrier()
    pltpu.sync_copy(acc.at[pl.ds(s * ROWS_T, ROWS_T)], tbuf)
    pltpu.sync_copy(tbuf, out_hbm.at[c, pl.ds(s * ROWS_T, ROWS_T), :])


def _mm1_body(x_ref, w_ref, o_ref):
    o_ref[pl.ds(0, N), :] = jnp.dot(x_ref[...], w_ref[...],
                                    preferred_element_type=jnp.float32)
    o_ref[pl.ds(N, NP - N), :] = jnp.zeros((NP - N, F), jnp.float32)


_mm1 = pl.pallas_call(
    _mm1_body, out_shape=jax.ShapeDtypeStruct((NP, F), jnp.float32))


def _mid_body(aggp_ref, y_ref, dinv_ref, b_ref, w_ref, o_ref):
    agg = aggp_ref[0, :N, :] + aggp_ref[1, :N, :]
    dv = dinv_ref[:N, :]
    h = dv * (agg + y_ref[:N, :]) + b_ref[...]
    h = jnp.maximum(h, 0.0)
    o_ref[...] = dv * jnp.dot(h, w_ref[...],
                              preferred_element_type=jnp.float32)


_mid = pl.pallas_call(
    _mid_body, out_shape=jax.ShapeDtypeStruct((N, F), jnp.float32))


def _mid2_body(aggp_ref, y_ref, dinv_ref, b_ref, o_ref):
    agg = aggp_ref[0, :N, :] + aggp_ref[1, :N, :]
    dv = dinv_ref[:N, :]
    h = dv * (agg + y_ref[...]) + b_ref[...]
    o_ref[...] = dv * jnp.maximum(h, 0.0)


_mid2 = pl.pallas_call(
    _mid2_body, out_shape=jax.ShapeDtypeStruct((N, F), jnp.float32))


def _final_body(aggp_ref, y_ref, dinv_ref, b_ref, w_ref, o_ref):
    agg = aggp_ref[0, :N, :] + aggp_ref[1, :N, :]
    a = dinv_ref[:N, :] * (agg + y_ref[...])
    o = jnp.dot(a, w_ref[...], preferred_element_type=jnp.float32) + b_ref[...]
    m = jnp.max(o, axis=1, keepdims=True)
    lse = jnp.log(jnp.sum(jnp.exp(o - m), axis=1, keepdims=True)) + m
    o_ref[...] = o - lse


_final = pl.pallas_call(
    _final_body, out_shape=jax.ShapeDtypeStruct((N, NCLS), jnp.float32))


def kernel(x, edge_index, edge_weight, W1, b1, W3, b3, W2, b2):
    pad = EPAD - E
    if pad:
        zi = jnp.zeros((pad,), edge_index.dtype)
        row_f = jnp.concatenate([edge_index[0], zi])
        col_f = jnp.concatenate([edge_index[1], zi])
        ew_f = jnp.concatenate([edge_weight,
                                jnp.zeros((pad,), edge_weight.dtype)])
    else:
        row_f, col_f, ew_f = edge_index[0], edge_index[1], edge_weight
    row3 = row_f.reshape(NW, NCHUNK, CH)
    col3 = col_f.reshape(NW, NCHUNK, CH)
    ew3 = ew_f.reshape(NW, NCHUNK, CH)
    xw1 = _mm1(x, W1)
    a1, dinv16, y1 = _layer1_kernel(xw1, row3, col3, ew3)
    y2 = _mid(a1, y1, dinv16, b1.reshape(1, F), W3)
    a2 = _agg_kernel(y2, row3, col3, ew3)
    y3 = _mid2(a2, y2, dinv16, b3.reshape(1, F))
    a3 = _agg_kernel(y3, row3, col3, ew3)
    return _final(a3, y3, dinv16, b2.reshape(1, NCLS), W2)
